# baseline (speedup 1.0000x reference)
import jax
import jax.numpy as jnp
from jax.experimental import pallas as pl
from jax.experimental.pallas import tpu as pltpu
import functools

_S = 25
_NA = 8
_GROUP_SLOTS = (1, 4, 9, 25, 2, 3, 6, 25)


def _body(ti_ref, p_ref, ans_ref, out_ref, acc_ref, *, bb, nsteps, btotal):
    i = pl.program_id(0)

    @pl.when(i == 0)
    def _init():
        acc_ref[...] = jnp.zeros_like(acc_ref)

    ti = ti_ref[...]
    p = p_ref[...]
    ans = ans_ref[...]

    idx = jnp.clip(ti - _NA, 0, _NA - 1)

    target = jnp.zeros((bb, 101), jnp.float32)
    sel = []
    for a in range(_NA):
        sa = idx == a
        sel.append(sa)
        target = target + jnp.where(sa, ans[:, a, :], 0.0)

    tg = target[:, 0:1]
    g = jnp.clip(tg.astype(jnp.int32), 0, _NA - 1)
    counts = jnp.zeros((bb, 1), jnp.int32)
    for k in range(_NA):
        counts = counts + jnp.where(g == k, _GROUP_SLOTS[k], 0)

    lane = jax.lax.broadcasted_iota(jnp.int32, (bb, 101), 1)
    rm_f = ((lane >= 1) & (lane <= counts)).astype(jnp.float32)

    r_idx = jax.lax.broadcasted_iota(jnp.int32, (75, _S), 0) // 3
    k_idx = jax.lax.broadcasted_iota(jnp.int32, (75, _S), 1)
    et = (r_idx == k_idx).astype(jnp.float32)

    d = (target != p).astype(jnp.float32)
    ham_range = jnp.sum(d * rm_f, axis=1, keepdims=True)
    d3 = jax.lax.dot(d[:, 26:101], et, preferred_element_type=jnp.float32)
    slot_f = (target[:, 1:26] > 0).astype(jnp.float32)
    ham_props = jnp.sum(slot_f * d3, axis=1, keepdims=True)

    masked_diff = ham_range + ham_props
    ham_sum = masked_diff + d[:, 0:1]
    acc_same_b = (masked_diff == 0).astype(jnp.float32)
    hamf_b = jnp.sum(d, axis=1, keepdims=True)
    fm_sum = counts.astype(jnp.float32) + 3.0 * jnp.sum(
        slot_f, axis=1, keepdims=True)
    hams_b = ham_sum / jnp.maximum(fm_sum, 1.0)

    n_zero = jnp.zeros((bb, 1), jnp.float32)
    tz = jnp.zeros((bb, 1), jnp.bool_)
    for a in range(_NA):
        ans_a = ans[:, a, :]
        da = (ans_a != p).astype(jnp.float32)
        hr_a = jnp.sum(da * rm_f, axis=1, keepdims=True)
        da3 = jax.lax.dot(da[:, 26:101], et,
                          preferred_element_type=jnp.float32)
        asm_f = (ans_a[:, 1:26] > 0).astype(jnp.float32)
        sums_a = hr_a + jnp.sum(asm_f * da3, axis=1, keepdims=True)
        zero_a = sums_a == 0
        n_zero = n_zero + zero_a.astype(jnp.float32)
        tz = tz | (zero_a & sel[a])

    matches_b = tz.astype(jnp.float32)
    once_b = (tz & (n_zero == 1)).astype(jnp.float32)

    lane128 = jax.lax.broadcasted_iota(jnp.int32, (1, 128), 1)
    partial = (
        jnp.sum(acc_same_b) * (lane128 == 0)
        + jnp.sum(hamf_b) * (lane128 == 1)
        + jnp.sum(ham_sum) * (lane128 == 2)
        + jnp.sum(fm_sum) * (lane128 == 3)
        + jnp.sum(hams_b) * (lane128 == 4)
        + jnp.sum(matches_b) * (lane128 == 5)
        + jnp.sum(once_b) * (lane128 == 6)
    )
    acc_ref[...] = acc_ref[...] + partial

    @pl.when(i == nsteps - 1)
    def _fin():
        acc = acc_ref[...]
        bf = jnp.float32(btotal)
        s_ham = jnp.sum(acc * (lane128 == 2))
        s_fm = jnp.sum(acc * (lane128 == 3))
        out_ref[...] = (
            acc * ((lane128 == 0) | (lane128 == 1) | (lane128 == 2)
                   | (lane128 == 4) | (lane128 == 5) | (lane128 == 6)) / bf
            + (1.0 - s_ham / (s_fm + bf)) * (lane128 == 7)
        )


def kernel(target_index, predict, contexts):
    b = predict.shape[0]
    bb = 512
    nsteps = b // bb
    ti = target_index.astype(jnp.int32)
    out = pl.pallas_call(
        functools.partial(_body, bb=bb, nsteps=nsteps, btotal=b),
        grid=(nsteps,),
        in_specs=[
            pl.BlockSpec((bb, 1), lambda i: (i, 0)),
            pl.BlockSpec((bb, 101), lambda i: (i, 0)),
            pl.BlockSpec((bb, _NA, 101), lambda i: (i, 1, 0)),
        ],
        out_specs=pl.BlockSpec((1, 128), lambda i: (0, 0)),
        out_shape=jax.ShapeDtypeStruct((1, 128), jnp.float32),
        scratch_shapes=[pltpu.VMEM((1, 128), jnp.float32)],
    )(ti, predict, contexts)
    return jnp.stack([out[0, 0], out[0, 1], out[0, 7], out[0, 2],
                      out[0, 4], out[0, 5], out[0, 6]])
